# baseline (speedup 1.0000x reference)
import functools

import jax
import jax.numpy as jnp
from jax import lax
from jax.experimental import pallas as pl
from jax.experimental.pallas import tpu as pltpu
from jax.experimental.pallas import tpu_sc as plsc

B = 4
NUM_FEAT = 768
D1 = 24
D2 = 24
NMEM = 1024
MLEN = 16
GROUPS = NUM_FEAT // MLEN
NROWS = B * D1 * D2
NCHUNKS = NROWS * GROUPS

NCORES = 2
NSUB = 16
NW = NCORES * NSUB
CPW = NCHUNKS // NW
SUBB = 6
CPB = CPW // SUBB
LANES = 16

RB = 384
NSTEP = NROWS // RB


def _mnorm_block(mem_ref, mnorm_ref):
    mem = mem_ref[...]
    nrm = jnp.sqrt(jnp.sum(mem * mem, axis=1, keepdims=True))
    mnorm_ref[...] = mem / jnp.maximum(nrm, 1e-12)


SPATIAL = D1 * D2
NBLK = B * GROUPS
UNROLL = 2


def _argmax_block(x_ref, mnorm_ref, idx_ref):
    mnorm = mnorm_ref[...]
    cols = []
    for u in range(UNROLL):
        X = x_ref[u]
        xnrm = jnp.sqrt(jnp.sum(X * X, axis=0, keepdims=True))
        Xn = X / jnp.maximum(xnrm, 1e-12)
        sim = lax.dot_general(
            Xn, mnorm, (((0,), (1,)), ((), ())),
            preferred_element_type=jnp.float32)
        cols.append(jnp.argmax(sim, axis=1).reshape(SPATIAL, 1))
    idx_ref[0] = jnp.concatenate(cols, axis=1)


def _tc_indices(xr, memory):
    mnorm = pl.pallas_call(
        _mnorm_block,
        out_shape=jax.ShapeDtypeStruct((NMEM, MLEN), jnp.float32),
    )(memory)
    return pl.pallas_call(
        _argmax_block,
        grid=(NBLK // UNROLL,),
        in_specs=[
            pl.BlockSpec((UNROLL, MLEN, SPATIAL), lambda i: (i, 0, 0)),
            pl.BlockSpec((NMEM, MLEN), lambda i: (0, 0)),
        ],
        out_specs=pl.BlockSpec((1, SPATIAL, UNROLL), lambda i: (i, 0, 0)),
        out_shape=jax.ShapeDtypeStruct((NBLK // UNROLL, SPATIAL, UNROLL),
                                       jnp.int32),
    )(xr, mnorm)


@functools.partial(
    pl.kernel,
    mesh=plsc.VectorSubcoreMesh(core_axis_name="c", subcore_axis_name="s"),
    out_type=jax.ShapeDtypeStruct((NCHUNKS * MLEN,), jnp.float32),
    scratch_types=[
        pltpu.VMEM((NMEM * MLEN,), jnp.float32),
        pltpu.VMEM((CPW,), jnp.int32),
        pltpu.VMEM((CPB * MLEN,), jnp.float32),
    ],
    compiler_params=pltpu.CompilerParams(needs_layout_passes=False),
)
def _sc_gather(table_hbm, idx_hbm, out_hbm, table_v, idx_v, out_v):
    wid = lax.axis_index("s") * NCORES + lax.axis_index("c")
    base = wid * CPW
    pltpu.sync_copy(table_hbm, table_v)
    pltpu.sync_copy(idx_hbm.at[pl.ds(base, CPW)], idx_v)
    posv = lax.broadcasted_iota(jnp.int32, (LANES,), 0) * MLEN

    for bt in range(SUBB):
        def j_body(j, carry):
            civ = idx_v[pl.ds(bt * CPB + j * LANES, LANES)]
            pos = posv + j * (LANES * MLEN)
            cbase = civ * MLEN
            for k in range(MLEN):
                g = plsc.load_gather(table_v, [cbase + k])
                plsc.store_scatter(out_v, [pos + k], g)
            return carry

        lax.fori_loop(0, CPB // LANES, j_body, 0)
        pltpu.sync_copy(
            out_v, out_hbm.at[pl.ds((base + bt * CPB) * MLEN, CPB * MLEN)])


@functools.partial(jax.jit, static_argnames=())
def kernel(x, memory):
    xr = x.reshape(NBLK, MLEN, SPATIAL)
    idx = _tc_indices(xr, memory)
    idxp = (idx.reshape(B, GROUPS // UNROLL, SPATIAL, UNROLL)
            .transpose(0, 2, 1, 3).reshape(-1))
    out = _sc_gather(memory.reshape(-1), idxp)
    out = out.reshape(B, D1, D2, NUM_FEAT)
    return jnp.transpose(out, (0, 3, 1, 2))


# speedup vs baseline: 1.0743x; 1.0743x over previous
import functools

import jax
import jax.numpy as jnp
from jax import lax
from jax.experimental import pallas as pl
from jax.experimental.pallas import tpu as pltpu
from jax.experimental.pallas import tpu_sc as plsc

B = 4
NUM_FEAT = 768
D1 = 24
D2 = 24
NMEM = 1024
MLEN = 16
GROUPS = NUM_FEAT // MLEN
NROWS = B * D1 * D2
NCHUNKS = NROWS * GROUPS

NCORES = 2
NSUB = 16
NW = NCORES * NSUB
CPW = NCHUNKS // NW
SUBB = 6
CPB = CPW // SUBB
LANES = 16

RB = 384
NSTEP = NROWS // RB


def _mnorm_block(mem_ref, mnorm_ref):
    mem = mem_ref[...]
    nrm = jnp.sqrt(jnp.sum(mem * mem, axis=1, keepdims=True))
    mnorm_ref[...] = mem / jnp.maximum(nrm, 1e-12)


SPATIAL = D1 * D2
NBLK = B * GROUPS
UNROLL = 2


def _argmax_block(x_ref, mnorm_ref, idx_ref):
    mnorm = mnorm_ref[...]
    for u in range(UNROLL):
        X = x_ref[u]
        xnrm = jnp.sqrt(jnp.sum(X * X, axis=0, keepdims=True))
        Xn = X / jnp.maximum(xnrm, 1e-12)
        sim = lax.dot_general(
            Xn, mnorm, (((0,), (1,)), ((), ())),
            preferred_element_type=jnp.float32)
        idx_ref[u] = jnp.argmax(sim, axis=1).reshape(1, SPATIAL)


def _tc_indices(xr, memory):
    mnorm = pl.pallas_call(
        _mnorm_block,
        out_shape=jax.ShapeDtypeStruct((NMEM, MLEN), jnp.float32),
    )(memory)
    return pl.pallas_call(
        _argmax_block,
        grid=(NBLK // UNROLL,),
        in_specs=[
            pl.BlockSpec((UNROLL, MLEN, SPATIAL), lambda i: (i, 0, 0)),
            pl.BlockSpec((NMEM, MLEN), lambda i: (0, 0)),
        ],
        out_specs=pl.BlockSpec((UNROLL, 1, SPATIAL), lambda i: (i, 0, 0)),
        out_shape=jax.ShapeDtypeStruct((NBLK, 1, SPATIAL), jnp.int32),
    )(xr, mnorm)


@functools.partial(
    pl.kernel,
    mesh=plsc.VectorSubcoreMesh(core_axis_name="c", subcore_axis_name="s"),
    out_type=jax.ShapeDtypeStruct((NCHUNKS * MLEN,), jnp.float32),
    scratch_types=[
        pltpu.VMEM((NMEM * MLEN,), jnp.float32),
        pltpu.VMEM((CPW,), jnp.int32),
        pltpu.VMEM((CPB * MLEN,), jnp.float32),
    ],
    compiler_params=pltpu.CompilerParams(needs_layout_passes=False),
)
def _sc_gather(table_hbm, idx_hbm, out_hbm, table_v, idx_v, out_v):
    wid = lax.axis_index("s") * NCORES + lax.axis_index("c")
    base = wid * CPW
    pltpu.sync_copy(table_hbm, table_v)
    pltpu.sync_copy(idx_hbm.at[pl.ds(base, CPW)], idx_v)
    posv = lax.broadcasted_iota(jnp.int32, (LANES,), 0) * MLEN

    for bt in range(SUBB):
        def j_body(j, carry):
            civ = idx_v[pl.ds(bt * CPB + j * LANES, LANES)]
            pos = posv + j * (LANES * MLEN)
            cbase = civ * MLEN
            for k in range(MLEN):
                g = plsc.load_gather(table_v, [cbase + k])
                plsc.store_scatter(out_v, [pos + k], g)
            return carry

        lax.fori_loop(0, CPB // LANES, j_body, 0)
        pltpu.sync_copy(
            out_v, out_hbm.at[pl.ds((base + bt * CPB) * MLEN, CPB * MLEN)])


@functools.partial(jax.jit, static_argnames=())
def kernel(x, memory):
    xr = x.reshape(NBLK, MLEN, SPATIAL)
    idx = _tc_indices(xr, memory)
    idxp = idx.reshape(B, GROUPS, SPATIAL).transpose(0, 2, 1).reshape(-1)
    out = _sc_gather(memory.reshape(-1), idxp)
    out = out.reshape(B, D1, D2, NUM_FEAT)
    return jnp.transpose(out, (0, 3, 1, 2))


# speedup vs baseline: 1.0789x; 1.0043x over previous
import functools

import jax
import jax.numpy as jnp
from jax import lax
from jax.experimental import pallas as pl
from jax.experimental.pallas import tpu as pltpu
from jax.experimental.pallas import tpu_sc as plsc

B = 4
NUM_FEAT = 768
D1 = 24
D2 = 24
NMEM = 1024
MLEN = 16
GROUPS = NUM_FEAT // MLEN
NROWS = B * D1 * D2
NCHUNKS = NROWS * GROUPS

NCORES = 2
NSUB = 16
NW = NCORES * NSUB
CPW = NCHUNKS // NW
SUBB = 2
CPB = CPW // SUBB
LANES = 16

RB = 384
NSTEP = NROWS // RB


def _mnorm_block(mem_ref, mnorm_ref):
    mem = mem_ref[...]
    nrm = jnp.sqrt(jnp.sum(mem * mem, axis=1, keepdims=True))
    mnorm_ref[...] = mem / jnp.maximum(nrm, 1e-12)


SPATIAL = D1 * D2
NBLK = B * GROUPS
UNROLL = 2


def _argmax_block(x_ref, mnorm_ref, idx_ref):
    mnorm = mnorm_ref[...]
    for u in range(UNROLL):
        X = x_ref[u]
        xnrm = jnp.sqrt(jnp.sum(X * X, axis=0, keepdims=True))
        Xn = X / jnp.maximum(xnrm, 1e-12)
        sim = lax.dot_general(
            Xn, mnorm, (((0,), (1,)), ((), ())),
            preferred_element_type=jnp.float32)
        idx_ref[u] = jnp.argmax(sim, axis=1).reshape(1, SPATIAL)


def _tc_indices(xr, memory):
    mnorm = pl.pallas_call(
        _mnorm_block,
        out_shape=jax.ShapeDtypeStruct((NMEM, MLEN), jnp.float32),
    )(memory)
    return pl.pallas_call(
        _argmax_block,
        grid=(NBLK // UNROLL,),
        in_specs=[
            pl.BlockSpec((UNROLL, MLEN, SPATIAL), lambda i: (i, 0, 0)),
            pl.BlockSpec((NMEM, MLEN), lambda i: (0, 0)),
        ],
        out_specs=pl.BlockSpec((UNROLL, 1, SPATIAL), lambda i: (i, 0, 0)),
        out_shape=jax.ShapeDtypeStruct((NBLK, 1, SPATIAL), jnp.int32),
    )(xr, mnorm)


@functools.partial(
    pl.kernel,
    mesh=plsc.VectorSubcoreMesh(core_axis_name="c", subcore_axis_name="s"),
    out_type=jax.ShapeDtypeStruct((NCHUNKS * MLEN,), jnp.float32),
    scratch_types=[
        pltpu.VMEM((NMEM * MLEN,), jnp.float32),
        pltpu.VMEM((CPW,), jnp.int32),
        pltpu.VMEM((CPB * MLEN,), jnp.float32),
    ],
    compiler_params=pltpu.CompilerParams(needs_layout_passes=False),
)
def _sc_gather(table_hbm, idx_hbm, out_hbm, table_v, idx_v, out_v):
    wid = lax.axis_index("s") * NCORES + lax.axis_index("c")
    base = wid * CPW
    pltpu.sync_copy(table_hbm, table_v)
    pltpu.sync_copy(idx_hbm.at[pl.ds(base, CPW)], idx_v)
    posv = lax.broadcasted_iota(jnp.int32, (LANES,), 0) * MLEN

    for bt in range(SUBB):
        def j_body(j, carry):
            civ = idx_v[pl.ds(bt * CPB + j * LANES, LANES)]
            pos = posv + j * (LANES * MLEN)
            cbase = civ * MLEN
            for k in range(MLEN):
                g = plsc.load_gather(table_v, [cbase + k])
                plsc.store_scatter(out_v, [pos + k], g)
            return carry

        lax.fori_loop(0, CPB // LANES, j_body, 0)
        pltpu.sync_copy(
            out_v, out_hbm.at[pl.ds((base + bt * CPB) * MLEN, CPB * MLEN)])


@functools.partial(jax.jit, static_argnames=())
def kernel(x, memory):
    xr = x.reshape(NBLK, MLEN, SPATIAL)
    idx = _tc_indices(xr, memory)
    idxp = idx.reshape(B, GROUPS, SPATIAL).transpose(0, 2, 1).reshape(-1)
    out = _sc_gather(memory.reshape(-1), idxp)
    out = out.reshape(B, D1, D2, NUM_FEAT)
    return jnp.transpose(out, (0, 3, 1, 2))
